# Initial kernel scaffold; baseline (speedup 1.0000x reference)
import jax
import jax.numpy as jnp
from jax.experimental import pallas as pl


def kernel(x, user_matrix, item_matrix):
    raise NotImplementedError("write your pallas kernel here")


# speedup vs baseline: 10.5550x; 10.5550x over previous
import functools

import jax
import jax.numpy as jnp
from jax import lax
from jax.experimental import pallas as pl
from jax.experimental.pallas import tpu as pltpu
from jax.experimental.pallas import tpu_sc as plsc

_B = 16384
_L = 16


@functools.lru_cache(maxsize=None)
def _build(nc: int, ns: int):
    nw = nc * ns
    b_per_w = _B // nw
    n_chunks = b_per_w // _L
    mesh = plsc.VectorSubcoreMesh(core_axis_name="c", subcore_axis_name="s")

    @functools.partial(
        pl.kernel,
        mesh=mesh,
        out_type=jax.ShapeDtypeStruct((_B,), jnp.float32),
        scratch_types=[
            pltpu.VMEM((2 * b_per_w,), jnp.int32),
            pltpu.VMEM((b_per_w,), jnp.float32),
            pltpu.VMEM((_L,), jnp.float32),
        ],
    )
    def mf(x_hbm, p_hbm, out_hbm, x_v, out_v, p_v):
        wid = lax.axis_index("s") * nc + lax.axis_index("c")
        base = wid * b_per_w

        pltpu.sync_copy(p_hbm, p_v)
        pltpu.sync_copy(x_hbm.at[pl.ds(base, b_per_w)],
                        x_v.at[pl.ds(0, b_per_w)])
        pltpu.sync_copy(x_hbm.at[pl.ds(_B + base, b_per_w)],
                        x_v.at[pl.ds(b_per_w, b_per_w)])

        k16 = lax.iota(jnp.int32, _L)
        r = lax.shift_right_logical(k16, 2)
        c = jnp.bitwise_and(k16, 3)
        p16 = p_v[...]
        u0 = p16.at[r].get(mode="promise_in_bounds")
        u1 = p16.at[r + 4].get(mode="promise_in_bounds")
        i0 = p16.at[c + 8].get(mode="promise_in_bounds")
        i1 = p16.at[c + 12].get(mode="promise_in_bounds")
        tab = u0 * i0 + u1 * i1

        for j in range(n_chunks):
            us = x_v[pl.ds(j * _L, _L)]
            its = x_v[pl.ds(b_per_w + j * _L, _L)]
            flat = lax.shift_left(us, 2) + its
            out_v[pl.ds(j * _L, _L)] = tab.at[flat].get(
                mode="promise_in_bounds")

        pltpu.sync_copy(out_v, out_hbm.at[pl.ds(base, b_per_w)])

    return mf


def kernel(x, user_matrix, item_matrix):
    info = plsc.get_sparse_core_info()
    packed = jnp.concatenate(
        [user_matrix.T.reshape(-1), item_matrix.reshape(-1)]).astype(
            jnp.float32)
    xt = x.astype(jnp.int32).T.reshape(-1)
    return _build(info.num_cores, info.num_subcores)(xt, packed)
